# baseline (speedup 1.0000x reference)
import functools

import jax
import jax.numpy as jnp
from jax import lax
from jax.experimental import pallas as pl
from jax.experimental.pallas import tpu as pltpu
from jax.experimental.pallas import tpu_sc as plsc

_B = 64
_S = 8192
_DIM = 64
_NC = 2
_NS = 16
_NW = _NC * _NS
_BPW = _B // _NW
_CROWS = 256
_CPB = _S // _CROWS
_NCHUNK = _BPW * _CPB
_NBUF = 4
_AHEAD = 2

_mesh = plsc.VectorSubcoreMesh(
    core_axis_name="c", subcore_axis_name="s",
    num_cores=_NC, num_subcores=_NS)


@functools.partial(
    pl.kernel,
    out_type=jax.ShapeDtypeStruct((_B, _S, _DIM), jnp.float32),
    mesh=_mesh,
    scratch_types=(
        [pltpu.VMEM((1, _CROWS, _DIM), jnp.float32) for _ in range(_NBUF)]
        + [pltpu.SemaphoreType.DMA for _ in range(2 * _NBUF)]
    ),
)
def _sc_copy(src_hbm, dst_hbm, *scratch):
    bufs = scratch[:_NBUF]
    sin = scratch[_NBUF:2 * _NBUF]
    sout = scratch[2 * _NBUF:]
    wid = lax.axis_index("s") * _NC + lax.axis_index("c")
    base_b = wid * _BPW

    def slc(c):
        b, r = divmod(c, _CPB)
        return (pl.ds(base_b + b, 1), pl.ds(r * _CROWS, _CROWS))

    def in_copy(c):
        b, r = slc(c)
        return pltpu.make_async_copy(
            src_hbm.at[b, r], bufs[c % _NBUF], sin[c % _NBUF])

    def out_copy(c):
        b, r = slc(c)
        return pltpu.make_async_copy(
            bufs[c % _NBUF], dst_hbm.at[b, r], sout[c % _NBUF])

    for c in range(_AHEAD):
        in_copy(c).start()
    for c in range(_NCHUNK):
        in_copy(c).wait()
        out_copy(c).start()
        j = c + _AHEAD
        if j < _NCHUNK:
            r = j - _NBUF
            if r >= 0:
                out_copy(r).wait()
            in_copy(j).start()
    for c in range(_NCHUNK - _NBUF, _NCHUNK):
        out_copy(c).wait()


def kernel(tensor, pool):
    del pool
    return _sc_copy(tensor)


# speedup vs baseline: 1.0339x; 1.0339x over previous
import functools

import jax
import jax.numpy as jnp
from jax import lax
from jax.experimental import pallas as pl
from jax.experimental.pallas import tpu as pltpu
from jax.experimental.pallas import tpu_sc as plsc

_B = 64
_S = 8192
_DIM = 64
_NC = 2
_BPW = _B // _NC
_CB = 1
_NCHUNK = _BPW // _CB
_NBUF = 3
_AHEAD = 2

_mesh = plsc.ScalarSubcoreMesh(axis_name="c", num_cores=_NC)


@functools.partial(
    pl.kernel,
    out_type=jax.ShapeDtypeStruct((_B, _S, _DIM), jnp.float32),
    mesh=_mesh,
    scratch_types=(
        [pltpu.MemorySpace.VMEM_SHARED((_CB, _S, _DIM), jnp.float32)
         for _ in range(_NBUF)]
        + [pltpu.SemaphoreType.DMA for _ in range(2 * _NBUF)]
    ),
)
def _sc_copy(src_hbm, dst_hbm, *scratch):
    bufs = scratch[:_NBUF]
    sin = scratch[_NBUF:2 * _NBUF]
    sout = scratch[2 * _NBUF:]
    base_b = lax.axis_index("c") * _BPW

    def in_copy(c):
        return pltpu.make_async_copy(
            src_hbm.at[pl.ds(base_b + c * _CB, _CB)],
            bufs[c % _NBUF], sin[c % _NBUF])

    def out_copy(c):
        return pltpu.make_async_copy(
            bufs[c % _NBUF],
            dst_hbm.at[pl.ds(base_b + c * _CB, _CB)], sout[c % _NBUF])

    for c in range(_AHEAD):
        in_copy(c).start()
    for c in range(_NCHUNK):
        in_copy(c).wait()
        out_copy(c).start()
        j = c + _AHEAD
        if j < _NCHUNK:
            r = j - _NBUF
            if r >= 0:
                out_copy(r).wait()
            in_copy(j).start()
    for c in range(_NCHUNK - _NBUF, _NCHUNK):
        out_copy(c).wait()


def kernel(tensor, pool):
    del pool
    return _sc_copy(tensor)


# speedup vs baseline: 1.0693x; 1.0343x over previous
import jax
import jax.numpy as jnp
from jax.experimental import pallas as pl

_B = 64
_S = 8192
_DIM = 64
_BB = 2


def _copy_body(src_ref, dst_ref):
    dst_ref[...] = src_ref[...]


def kernel(tensor, pool):
    del pool
    return pl.pallas_call(
        _copy_body,
        grid=(_B // _BB,),
        in_specs=[pl.BlockSpec((_BB, _S, _DIM), lambda i: (i, 0, 0))],
        out_specs=pl.BlockSpec((_BB, _S, _DIM), lambda i: (i, 0, 0)),
        out_shape=jax.ShapeDtypeStruct((_B, _S, _DIM), tensor.dtype),
    )(tensor)
